# baseline (speedup 1.0000x reference)
import functools
import jax
import jax.numpy as jnp
from jax import lax
from jax.experimental import pallas as pl
from jax.experimental.pallas import tpu as pltpu
from jax.experimental.pallas import tpu_sc as plsc

N = 32
MAX_EDGES = 128
D = 256
NUM_REL = 128
B = 16
COL = N + 3 * MAX_EDGES


def _rsqrt16(x):
    i = plsc.bitcast(x, jnp.int32)
    y = plsc.bitcast(jnp.int32(0x5F3759DF) - (i >> 1), jnp.float32)
    for _ in range(3):
        y = y * (1.5 - 0.5 * x * y * y)
    return y


def _sc_prep_body(bt_ref, x32_ref, s_ref, g_ref,
                  col_v, xidx_v, g_v, s_v, deg_v, seg_v, act_v, sem):
    wid = lax.axis_index("s") * 2 + lax.axis_index("c")

    @pl.when(wid < B)
    def _():
        b = wid
        iota = lax.iota(jnp.int32, 16)
        ones16 = jnp.ones((16,), jnp.float32)

        pltpu.sync_copy(bt_ref.at[b], col_v)
        for c in range(2):
            xidx_v[pl.ds(c * 16, 16)] = col_v[pl.ds(c * 16, 16)]
        pltpu.async_copy(x32_ref.at[xidx_v], g_v, sem).wait()
        pltpu.sync_copy(g_v, g_ref.at[b])

        carry = jnp.int32(0)
        for c in range(8):
            base = N + 2 * MAX_EDGES + c * 16
            cur = plsc.load_gather(col_v, [base + iota])
            prev = plsc.load_gather(col_v, [jnp.maximum(base + iota - 1,
                                                        N + 2 * MAX_EDGES)])
            ch = (cur != prev).astype(jnp.int32)
            seg_v[pl.ds(c * 16, 16)] = jnp.cumsum(ch) + carry
            carry = carry + jnp.sum(ch)
        nseg = carry + 1

        def zbody(i, x):
            deg_v[pl.ds(i * 16, 16)] = jnp.zeros((16,), jnp.float32)
            return x
        lax.fori_loop(0, (NUM_REL * N) // 16, zbody, 0)

        lane0 = iota == 0

        def cbody(e, tot):
            e_vec = jnp.full((16,), e, jnp.int32)
            seg_e = plsc.load_gather(seg_v, [e_vec])
            dst_e = plsc.load_gather(col_v, [N + MAX_EDGES + e_vec])
            plsc.addupdate_scatter(deg_v, [seg_e * N + dst_e], ones16,
                                   mask=lane0)
            is0 = (dst_e == 0) & lane0
            plsc.store_scatter(act_v, [jnp.full((16,), tot, jnp.int32)],
                               e_vec, mask=is0)
            return tot + jnp.sum(is0.astype(jnp.int32))

        total = lax.fori_loop(0, MAX_EDGES, cbody, jnp.int32(0))

        g0 = [g_v[0, pl.ds(j * 16, 16)] for j in range(16)]
        nseg_vec = jnp.full((16,), nseg, jnp.int32)

        def rbody(r, x):
            r_vec = jnp.full((16,), r, jnp.int32)
            cnt0 = plsc.load_gather(deg_v, [r_vec * N])
            rs = _rsqrt16(1.0 + cnt0)
            beta = jnp.where(r_vec < nseg_vec, rs * rs, 0.0)
            for j in range(16):
                plsc.store_scatter(s_v, [r_vec, j * 16 + iota], beta * g0[j])
            return x
        lax.fori_loop(0, NUM_REL, rbody, 0)

        def ebody(t, x):
            t_vec = jnp.full((16,), t, jnp.int32)
            e_vec = plsc.load_gather(act_v, [t_vec])
            seg_e = plsc.load_gather(seg_v, [e_vec])
            src_e = plsc.load_gather(col_v, [N + e_vec])
            cd = plsc.load_gather(deg_v, [seg_e * N])
            cs = plsc.load_gather(deg_v, [seg_e * N + src_e])
            a = _rsqrt16((1.0 + cd) * (1.0 + cs))
            for j in range(16):
                gc = plsc.load_gather(g_v, [src_e, j * 16 + iota])
                plsc.addupdate_scatter(s_v, [seg_e, j * 16 + iota], a * gc)
            return x
        lax.fori_loop(0, total, ebody, 0)

        pltpu.sync_copy(s_v, s_ref.at[b])


def _sc_prep(bt_t, x32):
    mesh = plsc.VectorSubcoreMesh(core_axis_name="c", subcore_axis_name="s")
    return pl.kernel(
        _sc_prep_body,
        mesh=mesh,
        compiler_params=pltpu.CompilerParams(needs_layout_passes=False),
        out_type=[
            jax.ShapeDtypeStruct((B, NUM_REL, D), jnp.float32),
            jax.ShapeDtypeStruct((B, N, D), jnp.float32),
        ],
        scratch_types=[
            pltpu.VMEM((COL,), jnp.int32),
            pltpu.VMEM((N,), jnp.int32),
            pltpu.VMEM((N, D), jnp.float32),
            pltpu.VMEM((NUM_REL, D), jnp.float32),
            pltpu.VMEM((NUM_REL * N,), jnp.float32),
            pltpu.VMEM((MAX_EDGES,), jnp.int32),
            pltpu.VMEM((MAX_EDGES,), jnp.int32),
            pltpu.SemaphoreType.DMA,
        ],
    )(bt_t, x32)


def _contract_body(s_ref, wr_ref, g_ref, ugw_ref, g0_ref, w0_ref, u_ref,
                   x1_ref, acc_c_ref, acc_n_ref, *, kblocks, ng_blocks):
    k = pl.program_id(0)

    @pl.when(k == 0)
    def _():
        acc_c_ref[...] = jnp.zeros_like(acc_c_ref)
        acc_n_ref[...] = jnp.zeros_like(acc_n_ref)

    acc_c_ref[...] += jnp.dot(s_ref[...], wr_ref[...],
                              preferred_element_type=jnp.float32)

    @pl.when(k < ng_blocks)
    def _():
        acc_n_ref[...] += jnp.dot(g_ref[...], ugw_ref[...],
                                  preferred_element_type=jnp.float32)

    @pl.when(k == kblocks - 1)
    def _():
        p0 = acc_c_ref[...] + jnp.dot(g0_ref[...], w0_ref[...],
                                      preferred_element_type=jnp.float32)
        ng = acc_n_ref[...]
        u_mat = u_ref[...]
        m = jnp.zeros((1, D), jnp.float32)
        rows = []
        for b in range(B):
            u = jax.nn.sigmoid(ng[b:b + 1] + jnp.dot(
                m.astype(jnp.bfloat16), u_mat,
                preferred_element_type=jnp.float32))
            m = u * p0[b:b + 1] + (1.0 - u) * m
            rows.append(jnp.maximum(m, 0.0))
        x1_ref[...] = jnp.concatenate(rows, axis=0)


def _head_body(x1_ref, w_ref, b_ref, out_ref):
    logits = jax.lax.dot_general(x1_ref[...], w_ref[...],
                                 (((1,), (1,)), ((), ())),
                                 preferred_element_type=jnp.float32)
    logits = logits + b_ref[...]
    mx = jnp.max(logits, axis=1, keepdims=True)
    sh = logits - mx
    lse = jnp.log(jnp.sum(jnp.exp(sh), axis=1, keepdims=True))
    out_ref[...] = sh - lse


def kernel(batchinput_tensor, X, Wr, W_0, update_gate_W, update_gate_U,
           W_glob, b_glob, W_sense, b_sense):
    bf16 = jnp.bfloat16
    bt_t = batchinput_tensor.T.astype(jnp.int32)
    x32 = X[:N]

    s_all, g_all = _sc_prep(bt_t, x32)

    s_flat = s_all.reshape(B, NUM_REL * D).astype(bf16)
    g_flat = g_all.reshape(B, N * D).astype(bf16)
    g0 = g_all[:, 0, :].astype(bf16)

    kb = 1024
    kblocks = (NUM_REL * D) // kb
    ng_blocks = (N * D) // kb

    x1 = pl.pallas_call(
        functools.partial(_contract_body, kblocks=kblocks, ng_blocks=ng_blocks),
        grid=(kblocks,),
        in_specs=[
            pl.BlockSpec((B, kb), lambda k: (0, k)),
            pl.BlockSpec((kb, D), lambda k: (k, 0)),
            pl.BlockSpec((B, kb), lambda k: (0, jnp.minimum(k, ng_blocks - 1))),
            pl.BlockSpec((kb, D), lambda k: (jnp.minimum(k, ng_blocks - 1), 0)),
            pl.BlockSpec((B, D), lambda k: (0, 0)),
            pl.BlockSpec((D, D), lambda k: (0, 0)),
            pl.BlockSpec((D, D), lambda k: (0, 0)),
        ],
        out_specs=pl.BlockSpec((B, D), lambda k: (0, 0)),
        out_shape=jax.ShapeDtypeStruct((B, D), jnp.float32),
        scratch_shapes=[
            pltpu.VMEM((B, D), jnp.float32),
            pltpu.VMEM((B, D), jnp.float32),
        ],
    )(s_flat, Wr.reshape(NUM_REL * D, D).astype(bf16), g_flat,
      update_gate_W.astype(bf16), g0, W_0.astype(bf16),
      update_gate_U.astype(bf16))

    x1b = x1.astype(bf16)

    def head(w, bias):
        v = w.shape[0]
        return pl.pallas_call(
            _head_body,
            in_specs=[
                pl.BlockSpec((B, D), lambda: (0, 0)),
                pl.BlockSpec((v, D), lambda: (0, 0)),
                pl.BlockSpec((1, v), lambda: (0, 0)),
            ],
            out_specs=pl.BlockSpec((B, v), lambda: (0, 0)),
            out_shape=jax.ShapeDtypeStruct((B, v), jnp.float32),
        )(x1b, w.astype(bf16), bias.reshape(1, v))

    preds_g = head(W_glob, b_glob)
    preds_s = head(W_sense, b_sense)
    return preds_g, preds_s


# speedup vs baseline: 1.2440x; 1.2440x over previous
import functools
import jax
import jax.numpy as jnp
from jax import lax
from jax.experimental import pallas as pl
from jax.experimental.pallas import tpu as pltpu
from jax.experimental.pallas import tpu_sc as plsc

N = 32
MAX_EDGES = 128
D = 256
NUM_REL = 128
B = 16
COL = N + 3 * MAX_EDGES


def _rsqrt16(x):
    i = plsc.bitcast(x, jnp.int32)
    y = plsc.bitcast(jnp.int32(0x5F3759DF) - (i >> 1), jnp.float32)
    for _ in range(3):
        y = y * (1.5 - 0.5 * x * y * y)
    return y


def _sc_prep_body(bt_ref, x32_ref, s_ref, g_ref,
                  col_v, xidx_v, g_v, s_v, deg_v, seg_v, act_v, sem):
    wid = lax.axis_index("s") * 2 + lax.axis_index("c")

    @pl.when(wid < B)
    def _():
        b = wid
        iota = lax.iota(jnp.int32, 16)
        ones16 = jnp.ones((16,), jnp.float32)

        pltpu.sync_copy(bt_ref.at[b], col_v)
        for c in range(2):
            xidx_v[pl.ds(c * 16, 16)] = col_v[pl.ds(c * 16, 16)]
        pltpu.async_copy(x32_ref.at[xidx_v], g_v, sem).wait()
        pltpu.sync_copy(g_v, g_ref.at[b])

        carry = jnp.int32(0)
        for c in range(8):
            base = N + 2 * MAX_EDGES + c * 16
            cur = plsc.load_gather(col_v, [base + iota])
            prev = plsc.load_gather(col_v, [jnp.maximum(base + iota - 1,
                                                        N + 2 * MAX_EDGES)])
            ch = (cur != prev).astype(jnp.int32)
            seg_v[pl.ds(c * 16, 16)] = jnp.cumsum(ch) + carry
            carry = carry + jnp.sum(ch)
        nseg = carry + 1

        def zbody(i, x):
            deg_v[pl.ds(i * 16, 16)] = jnp.zeros((16,), jnp.float32)
            return x
        lax.fori_loop(0, (NUM_REL * N) // 16, zbody, 0)

        lane0 = iota == 0

        def cbody(e, tot):
            e_vec = jnp.full((16,), e, jnp.int32)
            seg_e = plsc.load_gather(seg_v, [e_vec])
            dst_e = plsc.load_gather(col_v, [N + MAX_EDGES + e_vec])
            plsc.addupdate_scatter(deg_v, [seg_e * N + dst_e], ones16,
                                   mask=lane0)
            is0 = (dst_e == 0) & lane0
            plsc.store_scatter(act_v, [jnp.full((16,), tot, jnp.int32)],
                               e_vec, mask=is0)
            return tot + jnp.sum(is0.astype(jnp.int32))

        total = lax.fori_loop(0, MAX_EDGES, cbody, jnp.int32(0))

        g0 = [g_v[0, pl.ds(j * 16, 16)] for j in range(16)]
        nseg_vec = jnp.full((16,), nseg, jnp.int32)

        def rbody(r, x):
            r_vec = jnp.full((16,), r, jnp.int32)
            cnt0 = plsc.load_gather(deg_v, [r_vec * N])
            rs = _rsqrt16(1.0 + cnt0)
            beta = jnp.where(r_vec < nseg_vec, rs * rs, 0.0)
            for j in range(16):
                plsc.store_scatter(s_v, [r_vec, j * 16 + iota], beta * g0[j])
            return x
        lax.fori_loop(0, NUM_REL, rbody, 0)

        def ebody(t, x):
            t_vec = jnp.full((16,), t, jnp.int32)
            e_vec = plsc.load_gather(act_v, [t_vec])
            seg_e = plsc.load_gather(seg_v, [e_vec])
            src_e = plsc.load_gather(col_v, [N + e_vec])
            cd = plsc.load_gather(deg_v, [seg_e * N])
            cs = plsc.load_gather(deg_v, [seg_e * N + src_e])
            a = _rsqrt16((1.0 + cd) * (1.0 + cs))
            for j in range(16):
                gc = plsc.load_gather(g_v, [src_e, j * 16 + iota])
                plsc.addupdate_scatter(s_v, [seg_e, j * 16 + iota], a * gc)
            return x
        lax.fori_loop(0, total, ebody, 0)

        pltpu.sync_copy(s_v, s_ref.at[b])


def _sc_prep(bt_t, x32):
    mesh = plsc.VectorSubcoreMesh(core_axis_name="c", subcore_axis_name="s")
    return pl.kernel(
        _sc_prep_body,
        mesh=mesh,
        compiler_params=pltpu.CompilerParams(needs_layout_passes=False),
        out_type=[
            jax.ShapeDtypeStruct((B, NUM_REL, D), jnp.float32),
            jax.ShapeDtypeStruct((B, N, D), jnp.float32),
        ],
        scratch_types=[
            pltpu.VMEM((COL,), jnp.int32),
            pltpu.VMEM((N,), jnp.int32),
            pltpu.VMEM((N, D), jnp.float32),
            pltpu.VMEM((NUM_REL, D), jnp.float32),
            pltpu.VMEM((NUM_REL * N,), jnp.float32),
            pltpu.VMEM((MAX_EDGES,), jnp.int32),
            pltpu.VMEM((MAX_EDGES,), jnp.int32),
            pltpu.SemaphoreType.DMA,
        ],
    )(bt_t, x32)


def _contract_body(s_ref, wr_ref, g_ref, ugw_ref, g0_ref, w0_ref, u_ref,
                   x1_ref, acc_c_ref, acc_n_ref, *, kblocks, ng_blocks):
    k = pl.program_id(0)

    @pl.when(k == 0)
    def _():
        acc_c_ref[...] = jnp.zeros_like(acc_c_ref)
        acc_n_ref[...] = jnp.zeros_like(acc_n_ref)

    bf16 = jnp.bfloat16
    acc_c_ref[...] += jnp.dot(s_ref[...].astype(bf16), wr_ref[...].astype(bf16),
                              preferred_element_type=jnp.float32)

    @pl.when(k < ng_blocks)
    def _():
        acc_n_ref[...] += jnp.dot(g_ref[...].astype(bf16),
                                  ugw_ref[...].astype(bf16),
                                  preferred_element_type=jnp.float32)

    @pl.when(k == kblocks - 1)
    def _():
        p0 = acc_c_ref[...] + jnp.dot(g0_ref[...].astype(bf16),
                                      w0_ref[...].astype(bf16),
                                      preferred_element_type=jnp.float32)
        ng = acc_n_ref[...]
        u_mat = u_ref[...].astype(bf16)
        m = jnp.zeros((1, D), jnp.float32)
        rows = []
        for b in range(B):
            u = jax.nn.sigmoid(ng[b:b + 1] + jnp.dot(
                m.astype(bf16), u_mat,
                preferred_element_type=jnp.float32))
            m = u * p0[b:b + 1] + (1.0 - u) * m
            rows.append(jnp.maximum(m, 0.0))
        x1_ref[...] = jnp.concatenate(rows, axis=0)


def _head_body(x1_ref, w_ref, b_ref, out_ref):
    bf16 = jnp.bfloat16
    logits = jax.lax.dot_general(x1_ref[...].astype(bf16),
                                 w_ref[...].astype(bf16),
                                 (((1,), (1,)), ((), ())),
                                 preferred_element_type=jnp.float32)
    logits = logits + b_ref[...]
    mx = jnp.max(logits, axis=1, keepdims=True)
    sh = logits - mx
    lse = jnp.log(jnp.sum(jnp.exp(sh), axis=1, keepdims=True))
    out_ref[...] = sh - lse


def kernel(batchinput_tensor, X, Wr, W_0, update_gate_W, update_gate_U,
           W_glob, b_glob, W_sense, b_sense):
    bt_t = batchinput_tensor.T.astype(jnp.int32)
    x32 = X[:N]

    s_all, g_all = _sc_prep(bt_t, x32)

    s_flat = s_all.reshape(B, NUM_REL * D)
    g_flat = g_all.reshape(B, N * D)
    g0 = g_all[:, 0, :]

    kb = 1024
    kblocks = (NUM_REL * D) // kb
    ng_blocks = (N * D) // kb

    x1 = pl.pallas_call(
        functools.partial(_contract_body, kblocks=kblocks, ng_blocks=ng_blocks),
        grid=(kblocks,),
        in_specs=[
            pl.BlockSpec((B, kb), lambda k: (0, k)),
            pl.BlockSpec((kb, D), lambda k: (k, 0)),
            pl.BlockSpec((B, kb), lambda k: (0, jnp.minimum(k, ng_blocks - 1))),
            pl.BlockSpec((kb, D), lambda k: (jnp.minimum(k, ng_blocks - 1), 0)),
            pl.BlockSpec((B, D), lambda k: (0, 0)),
            pl.BlockSpec((D, D), lambda k: (0, 0)),
            pl.BlockSpec((D, D), lambda k: (0, 0)),
        ],
        out_specs=pl.BlockSpec((B, D), lambda k: (0, 0)),
        out_shape=jax.ShapeDtypeStruct((B, D), jnp.float32),
        scratch_shapes=[
            pltpu.VMEM((B, D), jnp.float32),
            pltpu.VMEM((B, D), jnp.float32),
        ],
    )(s_flat, Wr.reshape(NUM_REL * D, D), g_flat, update_gate_W,
      g0, W_0, update_gate_U)

    def head(w, bias):
        v = w.shape[0]
        return pl.pallas_call(
            _head_body,
            in_specs=[
                pl.BlockSpec((B, D), lambda: (0, 0)),
                pl.BlockSpec((v, D), lambda: (0, 0)),
                pl.BlockSpec((1, v), lambda: (0, 0)),
            ],
            out_specs=pl.BlockSpec((B, v), lambda: (0, 0)),
            out_shape=jax.ShapeDtypeStruct((B, v), jnp.float32),
        )(x1, w, bias.reshape(1, v))

    preds_g = head(W_glob, b_glob)
    preds_s = head(W_sense, b_sense)
    return preds_g, preds_s


# speedup vs baseline: 1.2854x; 1.0332x over previous
import functools
import jax
import jax.numpy as jnp
from jax import lax
from jax.experimental import pallas as pl
from jax.experimental.pallas import tpu as pltpu
from jax.experimental.pallas import tpu_sc as plsc

N = 32
MAX_EDGES = 128
D = 256
NUM_REL = 128
B = 16
COL = N + 3 * MAX_EDGES


def _rsqrt16(x):
    i = plsc.bitcast(x, jnp.int32)
    y = plsc.bitcast(jnp.int32(0x5F3759DF) - (i >> 1), jnp.float32)
    for _ in range(3):
        y = y * (1.5 - 0.5 * x * y * y)
    return y


RH = NUM_REL // 2


def _sc_prep_body(bt_ref, x32_ref, s_ref, g_ref,
                  col_v, xidx_v, g_v, s_v, deg_v, seg_v, act_v, sem):
    wid = lax.axis_index("s") * 2 + lax.axis_index("c")
    b = wid // 2
    h = wid % 2
    lo = h * RH
    iota = lax.iota(jnp.int32, 16)
    ones16 = jnp.ones((16,), jnp.float32)

    pltpu.sync_copy(bt_ref.at[b], col_v)
    for c in range(2):
        xidx_v[pl.ds(c * 16, 16)] = col_v[pl.ds(c * 16, 16)]
    pltpu.async_copy(x32_ref.at[xidx_v], g_v, sem).wait()

    @pl.when(h == 0)
    def _():
        pltpu.sync_copy(g_v, g_ref.at[b])

    carry = jnp.int32(0)
    e0 = jnp.int32(0)
    for c in range(8):
        base = N + 2 * MAX_EDGES + c * 16
        cur = plsc.load_gather(col_v, [base + iota])
        prev = plsc.load_gather(col_v, [jnp.maximum(base + iota - 1,
                                                    N + 2 * MAX_EDGES)])
        ch = (cur != prev).astype(jnp.int32)
        segc = jnp.cumsum(ch) + carry
        seg_v[pl.ds(c * 16, 16)] = segc
        carry = carry + jnp.sum(ch)
        e0 = e0 + jnp.sum((segc < RH).astype(jnp.int32))
    nseg = carry + 1
    estart = h * e0
    eend = (1 - h) * e0 + h * MAX_EDGES

    def zbody(i, x):
        deg_v[pl.ds(i * 16, 16)] = jnp.zeros((16,), jnp.float32)
        return x
    lax.fori_loop(0, (RH * N) // 16, zbody, 0)

    lane0 = iota == 0
    lo_vec = jnp.full((16,), lo, jnp.int32)

    def cbody(e, tot):
        e_vec = jnp.full((16,), e, jnp.int32)
        seg_e = plsc.load_gather(seg_v, [e_vec]) - lo_vec
        dst_e = plsc.load_gather(col_v, [N + MAX_EDGES + e_vec])
        plsc.addupdate_scatter(deg_v, [seg_e * N + dst_e], ones16,
                               mask=lane0)
        is0 = (dst_e == 0) & lane0
        plsc.store_scatter(act_v, [jnp.full((16,), tot, jnp.int32)],
                           e_vec, mask=is0)
        return tot + jnp.sum(is0.astype(jnp.int32))

    total = lax.fori_loop(estart, eend, cbody, jnp.int32(0))

    g0 = [g_v[0, pl.ds(j * 16, 16)] for j in range(16)]
    nseg_vec = jnp.full((16,), nseg, jnp.int32)

    def rbody(r, x):
        r_vec = jnp.full((16,), r, jnp.int32)
        cnt0 = plsc.load_gather(deg_v, [r_vec * N])
        rs = _rsqrt16(1.0 + cnt0)
        beta = jnp.where(r_vec + lo_vec < nseg_vec, rs * rs, 0.0)
        for j in range(16):
            plsc.store_scatter(s_v, [r_vec, j * 16 + iota], beta * g0[j])
        return x
    lax.fori_loop(0, RH, rbody, 0)

    def ebody(t, x):
        t_vec = jnp.full((16,), t, jnp.int32)
        e_vec = plsc.load_gather(act_v, [t_vec])
        seg_e = plsc.load_gather(seg_v, [e_vec]) - lo_vec
        src_e = plsc.load_gather(col_v, [N + e_vec])
        cd = plsc.load_gather(deg_v, [seg_e * N])
        cs = plsc.load_gather(deg_v, [seg_e * N + src_e])
        a = _rsqrt16((1.0 + cd) * (1.0 + cs))
        for j in range(16):
            gc = plsc.load_gather(g_v, [src_e, j * 16 + iota])
            plsc.addupdate_scatter(s_v, [seg_e, j * 16 + iota], a * gc)
        return x
    lax.fori_loop(0, total, ebody, 0)

    pltpu.sync_copy(s_v, s_ref.at[b, pl.ds(lo, RH)])


def _sc_prep(bt_t, x32):
    mesh = plsc.VectorSubcoreMesh(core_axis_name="c", subcore_axis_name="s")
    return pl.kernel(
        _sc_prep_body,
        mesh=mesh,
        compiler_params=pltpu.CompilerParams(needs_layout_passes=False),
        out_type=[
            jax.ShapeDtypeStruct((B, NUM_REL, D), jnp.float32),
            jax.ShapeDtypeStruct((B, N, D), jnp.float32),
        ],
        scratch_types=[
            pltpu.VMEM((COL,), jnp.int32),
            pltpu.VMEM((N,), jnp.int32),
            pltpu.VMEM((N, D), jnp.float32),
            pltpu.VMEM((RH, D), jnp.float32),
            pltpu.VMEM((RH * N,), jnp.float32),
            pltpu.VMEM((MAX_EDGES,), jnp.int32),
            pltpu.VMEM((MAX_EDGES,), jnp.int32),
            pltpu.SemaphoreType.DMA,
        ],
    )(bt_t, x32)


def _contract_body(s_ref, wr_ref, g_ref, ugw_ref, g0_ref, w0_ref, u_ref,
                   x1_ref, acc_c_ref, acc_n_ref, *, kblocks, ng_blocks):
    k = pl.program_id(0)

    @pl.when(k == 0)
    def _():
        acc_c_ref[...] = jnp.zeros_like(acc_c_ref)
        acc_n_ref[...] = jnp.zeros_like(acc_n_ref)

    bf16 = jnp.bfloat16
    acc_c_ref[...] += jnp.dot(s_ref[...].astype(bf16), wr_ref[...].astype(bf16),
                              preferred_element_type=jnp.float32)

    @pl.when(k < ng_blocks)
    def _():
        acc_n_ref[...] += jnp.dot(g_ref[...].astype(bf16),
                                  ugw_ref[...].astype(bf16),
                                  preferred_element_type=jnp.float32)

    @pl.when(k == kblocks - 1)
    def _():
        p0 = acc_c_ref[...] + jnp.dot(g0_ref[...].astype(bf16),
                                      w0_ref[...].astype(bf16),
                                      preferred_element_type=jnp.float32)
        ng = acc_n_ref[...]
        u_mat = u_ref[...].astype(bf16)
        m = jnp.zeros((1, D), jnp.float32)
        rows = []
        for b in range(B):
            u = jax.nn.sigmoid(ng[b:b + 1] + jnp.dot(
                m.astype(bf16), u_mat,
                preferred_element_type=jnp.float32))
            m = u * p0[b:b + 1] + (1.0 - u) * m
            rows.append(jnp.maximum(m, 0.0))
        x1_ref[...] = jnp.concatenate(rows, axis=0)


def _head_body(x1_ref, w_ref, b_ref, out_ref):
    bf16 = jnp.bfloat16
    logits = jax.lax.dot_general(x1_ref[...].astype(bf16),
                                 w_ref[...].astype(bf16),
                                 (((1,), (1,)), ((), ())),
                                 preferred_element_type=jnp.float32)
    logits = logits + b_ref[...]
    mx = jnp.max(logits, axis=1, keepdims=True)
    sh = logits - mx
    lse = jnp.log(jnp.sum(jnp.exp(sh), axis=1, keepdims=True))
    out_ref[...] = sh - lse


def kernel(batchinput_tensor, X, Wr, W_0, update_gate_W, update_gate_U,
           W_glob, b_glob, W_sense, b_sense):
    bt_t = batchinput_tensor.T.astype(jnp.int32)
    x32 = X[:N]

    s_all, g_all = _sc_prep(bt_t, x32)

    s_flat = s_all.reshape(B, NUM_REL * D)
    g_flat = g_all.reshape(B, N * D)
    g0 = g_all[:, 0, :]

    kb = 1024
    kblocks = (NUM_REL * D) // kb
    ng_blocks = (N * D) // kb

    x1 = pl.pallas_call(
        functools.partial(_contract_body, kblocks=kblocks, ng_blocks=ng_blocks),
        grid=(kblocks,),
        in_specs=[
            pl.BlockSpec((B, kb), lambda k: (0, k)),
            pl.BlockSpec((kb, D), lambda k: (k, 0)),
            pl.BlockSpec((B, kb), lambda k: (0, jnp.minimum(k, ng_blocks - 1))),
            pl.BlockSpec((kb, D), lambda k: (jnp.minimum(k, ng_blocks - 1), 0)),
            pl.BlockSpec((B, D), lambda k: (0, 0)),
            pl.BlockSpec((D, D), lambda k: (0, 0)),
            pl.BlockSpec((D, D), lambda k: (0, 0)),
        ],
        out_specs=pl.BlockSpec((B, D), lambda k: (0, 0)),
        out_shape=jax.ShapeDtypeStruct((B, D), jnp.float32),
        scratch_shapes=[
            pltpu.VMEM((B, D), jnp.float32),
            pltpu.VMEM((B, D), jnp.float32),
        ],
    )(s_flat, Wr.reshape(NUM_REL * D, D), g_flat, update_gate_W,
      g0, W_0, update_gate_U)

    def head(w, bias):
        v = w.shape[0]
        return pl.pallas_call(
            _head_body,
            in_specs=[
                pl.BlockSpec((B, D), lambda: (0, 0)),
                pl.BlockSpec((v, D), lambda: (0, 0)),
                pl.BlockSpec((1, v), lambda: (0, 0)),
            ],
            out_specs=pl.BlockSpec((B, v), lambda: (0, 0)),
            out_shape=jax.ShapeDtypeStruct((B, v), jnp.float32),
        )(x1, w, bias.reshape(1, v))

    preds_g = head(W_glob, b_glob)
    preds_s = head(W_sense, b_sense)
    return preds_g, preds_s
